# baseline (speedup 1.0000x reference)
import functools

import jax
import jax.numpy as jnp
from jax import lax
from jax.experimental import pallas as pl
from jax.experimental.pallas import tpu as pltpu
from jax.experimental.pallas import tpu_sc as plsc

CAP = 1048576
N_SAMPLES = 16384
LEAF_BLK = 32
ROW_W = 128
N_BLOCKS = CAP // LEAF_BLK
N_ROWS = CAP // ROW_W
HEAP = 2 * N_BLOCKS
L = 16
NC, NS_SUB = 2, 16
NW = NC * NS_SUB
LEAF_PER_W = CAP // NW
BLK_PER_W = N_BLOCKS // NW
Q_PER_W = N_SAMPLES // NW
QCHUNK = 128
PITCH = 17
SPITCH = ROW_W + 1
SPAD_G = L * SPITCH

_MESH = plsc.VectorSubcoreMesh(
    core_axis_name="c", subcore_axis_name="s", num_cores=NC, num_subcores=NS_SUB
)
_PARAMS = pltpu.CompilerParams(
    needs_layout_passes=False, skip_device_barrier=True
)


def _wid():
    return lax.axis_index("s") * NC + lax.axis_index("c")


@functools.partial(
    pl.kernel,
    out_type=jax.ShapeDtypeStruct((HEAP,), jnp.float32),
    mesh=_MESH,
    compiler_params=_PARAMS,
    scratch_types=[
        pltpu.VMEM((LEAF_PER_W,), jnp.float32),
        pltpu.VMEM((BLK_PER_W // L * L * PITCH,), jnp.float32),
        pltpu.VMEM((BLK_PER_W,), jnp.float32),
        pltpu.VMEM((512,), jnp.float32),
        pltpu.VMEM((256,), jnp.float32),
        pltpu.VMEM((128,), jnp.float32),
        pltpu.VMEM((64,), jnp.float32),
        pltpu.VMEM((32,), jnp.float32),
        pltpu.VMEM((16,), jnp.float32),
        pltpu.SemaphoreType.DMA,
        pltpu.SemaphoreType.DMA,
    ],
)
def _build_kernel(prior_hbm, tree_hbm, chunk, tpad, l15, l14, l13, l12, l11,
                  l10, l9, semA, semB):
    w = _wid()
    iota = lax.iota(jnp.int32, L)
    half = LEAF_PER_W // 2
    dmaA = pltpu.async_copy(
        prior_hbm.at[pl.ds(w * LEAF_PER_W, half)], chunk.at[pl.ds(0, half)], semA
    )
    dmaB = pltpu.async_copy(
        prior_hbm.at[pl.ds(w * LEAF_PER_W + half, half)],
        chunk.at[pl.ds(half, half)],
        semB,
    )

    def sums(g_lo, g_hi):
        @plsc.parallel_loop(g_lo, g_hi, unroll=2)
        def blk_body(g):
            base = g * (L * LEAF_BLK)
            pad0 = g * (L * PITCH)
            for b in range(L):
                v0 = chunk[pl.ds(base + b * LEAF_BLK, L)]
                v1 = chunk[pl.ds(base + b * LEAF_BLK + L, L)]
                tpad[pl.ds(pad0 + b * PITCH, L)] = v0 + v1
            accs = [jnp.zeros((L,), jnp.float32) for _ in range(4)]
            for c in range(L):
                accs[c % 4] = accs[c % 4] + plsc.load_gather(
                    tpad, [pad0 + iota * PITCH + c]
                )
            l15[pl.ds(g * L, L)] = (accs[0] + accs[1]) + (accs[2] + accs[3])

    dmaA.wait()
    sums(0, BLK_PER_W // (2 * L))
    dmaB.wait()
    sums(BLK_PER_W // (2 * L), BLK_PER_W // L)

    def reduce_level(src, dst, n_dst):
        for g in range(n_dst // L):
            b = g * L
            ev = plsc.load_gather(src, [2 * (b + iota)])
            od = plsc.load_gather(src, [2 * (b + iota) + 1])
            dst[pl.ds(b, L)] = ev + od

    reduce_level(l15, l14, 512)
    reduce_level(l14, l13, 256)
    reduce_level(l13, l12, 128)
    reduce_level(l12, l11, 64)
    reduce_level(l11, l10, 32)
    reduce_level(l10, l9, 16)

    pltpu.sync_copy(l15, tree_hbm.at[pl.ds(N_BLOCKS + w * BLK_PER_W, BLK_PER_W)])
    pltpu.sync_copy(l14, tree_hbm.at[pl.ds(16384 + w * 512, 512)])
    pltpu.sync_copy(l13, tree_hbm.at[pl.ds(8192 + w * 256, 256)])
    pltpu.sync_copy(l12, tree_hbm.at[pl.ds(4096 + w * 128, 128)])
    pltpu.sync_copy(l11, tree_hbm.at[pl.ds(2048 + w * 64, 64)])
    pltpu.sync_copy(l10, tree_hbm.at[pl.ds(1024 + w * 32, 32)])
    pltpu.sync_copy(l9, tree_hbm.at[pl.ds(512 + w * 16, 16)])


@functools.partial(
    pl.kernel,
    out_type=jax.ShapeDtypeStruct((N_SAMPLES,), jnp.int32),
    mesh=_MESH,
    compiler_params=_PARAMS,
    scratch_types=[
        pltpu.VMEM((HEAP,), jnp.float32),
        pltpu.VMEM((Q_PER_W,), jnp.float32),
        pltpu.VMEM((Q_PER_W,), jnp.float32),
        pltpu.VMEM((Q_PER_W,), jnp.int32),
        pltpu.VMEM((Q_PER_W,), jnp.int32),
        pltpu.VMEM((2, QCHUNK, ROW_W), jnp.float32),
        pltpu.VMEM((QCHUNK // L * SPAD_G,), jnp.float32),
        pltpu.VMEM((Q_PER_W,), jnp.int32),
        pltpu.SemaphoreType.DMA,
        pltpu.SemaphoreType.DMA,
        pltpu.SemaphoreType.DMA,
    ],
)
def _sample_kernel(prior2d_hbm, u_hbm, tree_hbm, out_hbm,
                   heap, ubuf, rbuf, bbuf, rowidx, rows, spad, leafbuf,
                   sem0, sem1, sem_stage):
    w = _wid()
    pltpu.sync_copy(tree_hbm.at[pl.ds(512, 512)], heap.at[pl.ds(512, 512)])
    staging = pltpu.async_copy(
        tree_hbm.at[pl.ds(1024, HEAP - 1024)],
        heap.at[pl.ds(1024, HEAP - 1024)],
        sem_stage,
    )
    pltpu.sync_copy(u_hbm.at[pl.ds(w * Q_PER_W, Q_PER_W)], ubuf)
    iota = lax.iota(jnp.int32, L)

    for k in (8, 7, 6, 5, 4):
        n = 1 << k
        for gi in range(n // L):
            b = n + gi * L
            ev = plsc.load_gather(heap, [2 * (b + iota)])
            od = plsc.load_gather(heap, [2 * (b + iota) + 1])
            heap[pl.ds(b, L)] = ev + od
    ev = plsc.load_gather(heap, [2 * (8 + iota)])
    od = plsc.load_gather(heap, [2 * (8 + iota) + 1])
    cur = heap[pl.ds(8, L)]
    heap[pl.ds(8, L)] = jnp.where(iota < 8, ev + od, cur)
    for _ in range(3):
        ev = plsc.load_gather(heap, [2 * iota])
        od = plsc.load_gather(heap, [2 * iota + 1])
        cur = heap[pl.ds(0, L)]
        heap[pl.ds(0, L)] = jnp.where((iota >= 1) & (iota < 8), ev + od, cur)

    total = heap[pl.ds(0, L)][1]
    step = total * (1.0 / N_SAMPLES)
    qbase = w * Q_PER_W

    @plsc.parallel_loop(0, Q_PER_W // L, unroll=2)
    def desc_body(g):
        q = qbase + g * L + iota
        u = ubuf[pl.ds(g * L, L)]
        s = (q.astype(jnp.float32) + u) * step

        def lvl(_, nc):
            node, sv = nc
            left = 2 * node
            lv = plsc.load_gather(heap, [left])
            go_left = sv <= lv
            node = jnp.where(go_left, left, left + 1)
            sv = jnp.where(go_left, sv, sv - lv)
            return node, sv

        node, s = lax.fori_loop(0, 9, lvl, (jnp.ones((L,), jnp.int32), s))
        bbuf[pl.ds(g * L, L)] = node
        rbuf[pl.ds(g * L, L)] = s

    staging.wait()

    def desc2_chunk(c):
        @plsc.parallel_loop(c * (QCHUNK // L), (c + 1) * (QCHUNK // L), unroll=2)
        def desc2_body(g):
            def lvl(_, nc):
                node, sv = nc
                left = 2 * node
                lv = plsc.load_gather(heap, [left])
                go_left = sv <= lv
                node = jnp.where(go_left, left, left + 1)
                sv = jnp.where(go_left, sv, sv - lv)
                return node, sv

            node, s = lax.fori_loop(
                0, 6, lvl, (bbuf[pl.ds(g * L, L)], rbuf[pl.ds(g * L, L)])
            )
            b = node - N_BLOCKS
            bbuf[pl.ds(g * L, L)] = b
            rowidx[pl.ds(g * L, L)] = b >> 2
            rbuf[pl.ds(g * L, L)] = s

    sems = (sem0, sem1)

    def fire(c):
        return pltpu.async_copy(
            prior2d_hbm.at[rowidx.at[pl.ds(c * QCHUNK, QCHUNK)]],
            rows.at[c % 2],
            sems[c % 2],
        )

    def scan_chunk(c):
        buf = c % 2

        @plsc.parallel_loop(0, QCHUNK // L, unroll=2)
        def scan_group(g):
            q0 = c * QCHUNK + g * L
            pad0 = g * SPAD_G
            rvec = rbuf[pl.ds(q0, L)]
            bvec = bbuf[pl.ds(q0, L)]
            col0 = (bvec & 3) * LEAF_BLK

            def copy_row(l, carry):
                for k in range(ROW_W // L):
                    spad[pl.ds(pad0 + l * SPITCH + k * L, L)] = (
                        rows[buf, g * L + l, pl.ds(k * L, L)]
                    )
                return carry

            lax.fori_loop(0, L, copy_row, 0)

            def step_p(p, ac):
                acc, cnt = ac
                v = plsc.load_gather(spad, [pad0 + iota * SPITCH + col0 + p])
                acc = acc + v
                cnt = cnt + jnp.where(acc < rvec, 1, 0)
                return acc, cnt

            _, cnt = lax.fori_loop(
                0, LEAF_BLK, step_p,
                (jnp.zeros((L,), jnp.float32), jnp.zeros((L,), jnp.int32)),
            )
            leafbuf[pl.ds(q0, L)] = bvec * LEAF_BLK + jnp.minimum(cnt, LEAF_BLK - 1)

    n_chunks = Q_PER_W // QCHUNK
    pending = []
    desc2_chunk(0)
    pending.append(fire(0))
    desc2_chunk(1)
    pending.append(fire(1))
    for c in range(n_chunks):
        pending[c].wait()
        scan_chunk(c)
        if c + 2 < n_chunks:
            desc2_chunk(c + 2)
            pending.append(fire(c + 2))

    pltpu.sync_copy(leafbuf, out_hbm.at[pl.ds(qbase, Q_PER_W)])


def kernel(priorities, size):
    del size
    u = jax.random.uniform(
        jax.random.fold_in(jax.random.key(0), 1), (N_SAMPLES,), jnp.float32
    )
    tree = _build_kernel(priorities)
    return _sample_kernel(
        priorities.reshape(N_ROWS, ROW_W), u, tree
    )


# speedup vs baseline: 1.0249x; 1.0249x over previous
import functools

import jax
import jax.numpy as jnp
from jax import lax
from jax.experimental import pallas as pl
from jax.experimental.pallas import tpu as pltpu
from jax.experimental.pallas import tpu_sc as plsc

CAP = 1048576
N_SAMPLES = 16384
LEAF_BLK = 32
ROW_W = 128
N_BLOCKS = CAP // LEAF_BLK
N_ROWS = CAP // ROW_W
HEAP = 2 * N_BLOCKS
L = 16
NC, NS_SUB = 2, 16
NW = NC * NS_SUB
LEAF_PER_W = CAP // NW
BLK_PER_W = N_BLOCKS // NW
Q_PER_W = N_SAMPLES // NW
QCHUNK = 128
PITCH = 17
SPITCH = ROW_W + 1
SPAD_G = L * SPITCH

_MESH = plsc.VectorSubcoreMesh(
    core_axis_name="c", subcore_axis_name="s", num_cores=NC, num_subcores=NS_SUB
)
_PARAMS = pltpu.CompilerParams(needs_layout_passes=False)


def _wid():
    return lax.axis_index("s") * NC + lax.axis_index("c")


@functools.partial(
    pl.kernel,
    out_type=jax.ShapeDtypeStruct((HEAP,), jnp.float32),
    mesh=_MESH,
    compiler_params=_PARAMS,
    scratch_types=[
        pltpu.VMEM((LEAF_PER_W,), jnp.float32),
        pltpu.VMEM((BLK_PER_W // L * L * PITCH,), jnp.float32),
        pltpu.VMEM((BLK_PER_W,), jnp.float32),
        pltpu.VMEM((512,), jnp.float32),
        pltpu.VMEM((256,), jnp.float32),
        pltpu.VMEM((128,), jnp.float32),
        pltpu.VMEM((64,), jnp.float32),
        pltpu.VMEM((32,), jnp.float32),
        pltpu.VMEM((16,), jnp.float32),
        pltpu.SemaphoreType.DMA,
        pltpu.SemaphoreType.DMA,
    ],
)
def _build_kernel(prior_hbm, tree_hbm, chunk, tpad, l15, l14, l13, l12, l11,
                  l10, l9, semA, semB):
    w = _wid()
    iota = lax.iota(jnp.int32, L)
    half = LEAF_PER_W // 2
    dmaA = pltpu.async_copy(
        prior_hbm.at[pl.ds(w * LEAF_PER_W, half)], chunk.at[pl.ds(0, half)], semA
    )
    dmaB = pltpu.async_copy(
        prior_hbm.at[pl.ds(w * LEAF_PER_W + half, half)],
        chunk.at[pl.ds(half, half)],
        semB,
    )

    def sums(g_lo, g_hi):
        @plsc.parallel_loop(g_lo, g_hi, unroll=2)
        def blk_body(g):
            base = g * (L * LEAF_BLK)
            pad0 = g * (L * PITCH)
            for b in range(L):
                v0 = chunk[pl.ds(base + b * LEAF_BLK, L)]
                v1 = chunk[pl.ds(base + b * LEAF_BLK + L, L)]
                tpad[pl.ds(pad0 + b * PITCH, L)] = v0 + v1
            accs = [jnp.zeros((L,), jnp.float32) for _ in range(4)]
            for c in range(L):
                accs[c % 4] = accs[c % 4] + plsc.load_gather(
                    tpad, [pad0 + iota * PITCH + c]
                )
            l15[pl.ds(g * L, L)] = (accs[0] + accs[1]) + (accs[2] + accs[3])

    dmaA.wait()
    sums(0, BLK_PER_W // (2 * L))
    dmaB.wait()
    sums(BLK_PER_W // (2 * L), BLK_PER_W // L)

    def reduce_level(src, dst, n_dst):
        for g in range(n_dst // L):
            b = g * L
            ev = plsc.load_gather(src, [2 * (b + iota)])
            od = plsc.load_gather(src, [2 * (b + iota) + 1])
            dst[pl.ds(b, L)] = ev + od

    reduce_level(l15, l14, 512)
    reduce_level(l14, l13, 256)
    reduce_level(l13, l12, 128)
    reduce_level(l12, l11, 64)
    reduce_level(l11, l10, 32)
    reduce_level(l10, l9, 16)

    pltpu.sync_copy(l15, tree_hbm.at[pl.ds(N_BLOCKS + w * BLK_PER_W, BLK_PER_W)])
    pltpu.sync_copy(l14, tree_hbm.at[pl.ds(16384 + w * 512, 512)])
    pltpu.sync_copy(l13, tree_hbm.at[pl.ds(8192 + w * 256, 256)])
    pltpu.sync_copy(l12, tree_hbm.at[pl.ds(4096 + w * 128, 128)])
    pltpu.sync_copy(l11, tree_hbm.at[pl.ds(2048 + w * 64, 64)])
    pltpu.sync_copy(l10, tree_hbm.at[pl.ds(1024 + w * 32, 32)])
    pltpu.sync_copy(l9, tree_hbm.at[pl.ds(512 + w * 16, 16)])


@functools.partial(
    pl.kernel,
    out_type=jax.ShapeDtypeStruct((N_SAMPLES,), jnp.int32),
    mesh=_MESH,
    compiler_params=_PARAMS,
    scratch_types=[
        pltpu.VMEM((HEAP,), jnp.float32),
        pltpu.VMEM((Q_PER_W,), jnp.float32),
        pltpu.VMEM((Q_PER_W,), jnp.float32),
        pltpu.VMEM((Q_PER_W,), jnp.int32),
        pltpu.VMEM((Q_PER_W,), jnp.int32),
        pltpu.VMEM((2, QCHUNK, ROW_W), jnp.float32),
        pltpu.VMEM((QCHUNK // L * SPAD_G,), jnp.float32),
        pltpu.VMEM((Q_PER_W,), jnp.int32),
        pltpu.SemaphoreType.DMA,
        pltpu.SemaphoreType.DMA,
        pltpu.SemaphoreType.DMA,
    ],
)
def _sample_kernel(prior2d_hbm, u_hbm, tree_hbm, out_hbm,
                   heap, ubuf, rbuf, bbuf, rowidx, rows, spad, leafbuf,
                   sem0, sem1, sem_stage):
    w = _wid()
    pltpu.sync_copy(tree_hbm.at[pl.ds(512, 512)], heap.at[pl.ds(512, 512)])
    st1 = pltpu.async_copy(
        tree_hbm.at[pl.ds(1024, 15360)], heap.at[pl.ds(1024, 15360)], sem_stage
    )
    st2 = pltpu.async_copy(
        tree_hbm.at[pl.ds(16384, 16384)], heap.at[pl.ds(16384, 16384)], sem0
    )
    st3 = pltpu.async_copy(
        tree_hbm.at[pl.ds(32768, 32768)], heap.at[pl.ds(32768, 32768)], sem1
    )
    pltpu.sync_copy(u_hbm.at[pl.ds(w * Q_PER_W, Q_PER_W)], ubuf)
    iota = lax.iota(jnp.int32, L)

    for k in (8, 7, 6, 5, 4):
        n = 1 << k
        for gi in range(n // L):
            b = n + gi * L
            ev = plsc.load_gather(heap, [2 * (b + iota)])
            od = plsc.load_gather(heap, [2 * (b + iota) + 1])
            heap[pl.ds(b, L)] = ev + od
    ev = plsc.load_gather(heap, [2 * (8 + iota)])
    od = plsc.load_gather(heap, [2 * (8 + iota) + 1])
    cur = heap[pl.ds(8, L)]
    heap[pl.ds(8, L)] = jnp.where(iota < 8, ev + od, cur)
    for _ in range(3):
        ev = plsc.load_gather(heap, [2 * iota])
        od = plsc.load_gather(heap, [2 * iota + 1])
        cur = heap[pl.ds(0, L)]
        heap[pl.ds(0, L)] = jnp.where((iota >= 1) & (iota < 8), ev + od, cur)

    total = heap[pl.ds(0, L)][1]
    step = total * (1.0 / N_SAMPLES)
    qbase = w * Q_PER_W

    @plsc.parallel_loop(0, Q_PER_W // L, unroll=2)
    def desc_body(g):
        q = qbase + g * L + iota
        u = ubuf[pl.ds(g * L, L)]
        s = (q.astype(jnp.float32) + u) * step

        def lvl(_, nc):
            node, sv = nc
            left = 2 * node
            lv = plsc.load_gather(heap, [left])
            go_left = sv <= lv
            node = jnp.where(go_left, left, left + 1)
            sv = jnp.where(go_left, sv, sv - lv)
            return node, sv

        node, s = lax.fori_loop(0, 9, lvl, (jnp.ones((L,), jnp.int32), s))
        bbuf[pl.ds(g * L, L)] = node
        rbuf[pl.ds(g * L, L)] = s

    st1.wait()

    @plsc.parallel_loop(0, Q_PER_W // L, unroll=2)
    def desc_mid(g):
        def lvl(_, nc):
            node, sv = nc
            left = 2 * node
            lv = plsc.load_gather(heap, [left])
            go_left = sv <= lv
            node = jnp.where(go_left, left, left + 1)
            sv = jnp.where(go_left, sv, sv - lv)
            return node, sv

        node, s = lax.fori_loop(
            0, 4, lvl, (bbuf[pl.ds(g * L, L)], rbuf[pl.ds(g * L, L)])
        )
        bbuf[pl.ds(g * L, L)] = node
        rbuf[pl.ds(g * L, L)] = s

    st2.wait()
    st3.wait()

    def desc2_chunk(c):
        @plsc.parallel_loop(c * (QCHUNK // L), (c + 1) * (QCHUNK // L), unroll=2)
        def desc2_body(g):
            def lvl(_, nc):
                node, sv = nc
                left = 2 * node
                lv = plsc.load_gather(heap, [left])
                go_left = sv <= lv
                node = jnp.where(go_left, left, left + 1)
                sv = jnp.where(go_left, sv, sv - lv)
                return node, sv

            node, s = lax.fori_loop(
                0, 2, lvl, (bbuf[pl.ds(g * L, L)], rbuf[pl.ds(g * L, L)])
            )
            b = node - N_BLOCKS
            bbuf[pl.ds(g * L, L)] = b
            rowidx[pl.ds(g * L, L)] = b >> 2
            rbuf[pl.ds(g * L, L)] = s

    sems = (sem0, sem1)

    def fire(c):
        return pltpu.async_copy(
            prior2d_hbm.at[rowidx.at[pl.ds(c * QCHUNK, QCHUNK)]],
            rows.at[c % 2],
            sems[c % 2],
        )

    def scan_chunk(c):
        buf = c % 2

        @plsc.parallel_loop(0, QCHUNK // L, unroll=2)
        def scan_group(g):
            q0 = c * QCHUNK + g * L
            pad0 = g * SPAD_G
            rvec = rbuf[pl.ds(q0, L)]
            bvec = bbuf[pl.ds(q0, L)]
            col0 = (bvec & 3) * LEAF_BLK

            def copy_row(l, carry):
                for k in range(ROW_W // L):
                    spad[pl.ds(pad0 + l * SPITCH + k * L, L)] = (
                        rows[buf, g * L + l, pl.ds(k * L, L)]
                    )
                return carry

            lax.fori_loop(0, L, copy_row, 0)

            def step_p(p, ac):
                acc, cnt = ac
                v = plsc.load_gather(spad, [pad0 + iota * SPITCH + col0 + p])
                acc = acc + v
                cnt = cnt + jnp.where(acc < rvec, 1, 0)
                return acc, cnt

            _, cnt = lax.fori_loop(
                0, LEAF_BLK, step_p,
                (jnp.zeros((L,), jnp.float32), jnp.zeros((L,), jnp.int32)),
            )
            leafbuf[pl.ds(q0, L)] = bvec * LEAF_BLK + jnp.minimum(cnt, LEAF_BLK - 1)

    n_chunks = Q_PER_W // QCHUNK
    pending = []
    desc2_chunk(0)
    pending.append(fire(0))
    desc2_chunk(1)
    pending.append(fire(1))
    for c in range(n_chunks):
        pending[c].wait()
        scan_chunk(c)
        if c + 2 < n_chunks:
            desc2_chunk(c + 2)
            pending.append(fire(c + 2))

    pltpu.sync_copy(leafbuf, out_hbm.at[pl.ds(qbase, Q_PER_W)])


def kernel(priorities, size):
    del size
    u = jax.random.uniform(
        jax.random.fold_in(jax.random.key(0), 1), (N_SAMPLES,), jnp.float32
    )
    tree = _build_kernel(priorities)
    return _sample_kernel(
        priorities.reshape(N_ROWS, ROW_W), u, tree
    )
